# baseline (speedup 1.0000x reference)
import functools

import jax
import jax.numpy as jnp
from jax import lax
from jax.experimental import pallas as pl
from jax.experimental.pallas import tpu as pltpu
from jax.experimental.pallas import tpu_sc as plsc

NUM_CORES = 2
NUM_SUBCORES = 16
NUM_WORKERS = NUM_CORES * NUM_SUBCORES
LANES = 16
CHUNK = 128


def _dot_kernel(bpw, nch, d,
                uemb, memb, uidx_hbm, midx_hbm, out_hbm,
                uidx_v, midx_v, u_v, m_v, o_v, sem):
    wid = lax.axis_index("s") * NUM_CORES + lax.axis_index("c")
    base = wid * bpw

    pltpu.sync_copy(uidx_hbm.at[pl.ds(wid * nch, nch)], uidx_v)
    pltpu.sync_copy(midx_hbm.at[pl.ds(wid * nch, nch)], midx_v)

    copies = []
    for j in range(nch):
        copies.append(pltpu.async_copy(
            uemb.at[uidx_v.at[j]], u_v.at[pl.ds(j * CHUNK, CHUNK)], sem))
        copies.append(pltpu.async_copy(
            memb.at[midx_v.at[j]], m_v.at[pl.ds(j * CHUNK, CHUNK)], sem))
    for c in copies:
        c.wait()

    lane = lax.iota(jnp.int32, LANES)

    def group_body(g, carry):
        rows = lane + g * LANES
        acc = jnp.zeros((LANES,), jnp.float32)
        for dd in range(d):
            col = jnp.full((LANES,), dd, jnp.int32)
            uu = plsc.load_gather(u_v, [rows, col])
            mm = plsc.load_gather(m_v, [rows, col])
            acc = acc + uu * mm
        o_v[pl.ds(g * LANES, LANES)] = acc
        return carry

    lax.fori_loop(0, bpw // LANES, group_body, 0)

    pltpu.sync_copy(o_v, out_hbm.at[pl.ds(base, bpw)])


def kernel(x, user_emb, movie_emb):
    b = x.shape[1]
    d = user_emb.shape[1]
    bpw = b // NUM_WORKERS
    nch = bpw // CHUNK

    user_idx = x[0].reshape(NUM_WORKERS * nch, CHUNK)
    movie_idx = x[1].reshape(NUM_WORKERS * nch, CHUNK)

    mesh = plsc.VectorSubcoreMesh(core_axis_name="c", subcore_axis_name="s")
    run = functools.partial(
        pl.kernel, mesh=mesh,
        out_type=jax.ShapeDtypeStruct((b,), jnp.float32),
        scratch_types=[
            pltpu.VMEM((nch, CHUNK), jnp.int32),
            pltpu.VMEM((nch, CHUNK), jnp.int32),
            pltpu.VMEM((bpw, d), jnp.float32),
            pltpu.VMEM((bpw, d), jnp.float32),
            pltpu.VMEM((bpw,), jnp.float32),
            pltpu.SemaphoreType.DMA,
        ],
        compiler_params=pltpu.CompilerParams(
            use_tc_tiling_on_sc=False, needs_layout_passes=False),
    )(functools.partial(_dot_kernel, bpw, nch, d))
    return run(user_emb, movie_emb, user_idx, movie_idx)


# speedup vs baseline: 1.0025x; 1.0025x over previous
import functools

import jax
import jax.numpy as jnp
from jax import lax
from jax.experimental import pallas as pl
from jax.experimental.pallas import tpu as pltpu
from jax.experimental.pallas import tpu_sc as plsc

NUM_CORES = 2
NUM_SUBCORES = 16
NUM_WORKERS = NUM_CORES * NUM_SUBCORES
LANES = 16
CHUNK = 128


def _dot_kernel(bpw, nch, d,
                uemb, memb, uidx_hbm, midx_hbm, out_hbm,
                uidx_v, midx_v, u_v, m_v, o_v, sem):
    wid = lax.axis_index("s") * NUM_CORES + lax.axis_index("c")
    base = wid * bpw

    pltpu.sync_copy(uidx_hbm.at[pl.ds(base, bpw)], uidx_v)
    pltpu.sync_copy(midx_hbm.at[pl.ds(base, bpw)], midx_v)

    copies = []
    for j in range(nch):
        sl = pl.ds(j * CHUNK, CHUNK)
        copies.append(pltpu.async_copy(
            uemb.at[uidx_v.at[sl]], u_v.at[sl], sem))
        copies.append(pltpu.async_copy(
            memb.at[midx_v.at[sl]], m_v.at[sl], sem))
    for c in copies:
        c.wait()

    lane = lax.iota(jnp.int32, LANES)

    def group_body(g, carry):
        rows = lane + g * LANES
        acc = jnp.zeros((LANES,), jnp.float32)
        for dd in range(d):
            col = jnp.full((LANES,), dd, jnp.int32)
            uu = plsc.load_gather(u_v, [rows, col])
            mm = plsc.load_gather(m_v, [rows, col])
            acc = acc + uu * mm
        o_v[pl.ds(g * LANES, LANES)] = acc
        return carry

    lax.fori_loop(0, bpw // LANES, group_body, 0)

    pltpu.sync_copy(o_v, out_hbm.at[pl.ds(base, bpw)])


def kernel(x, user_emb, movie_emb):
    b = x.shape[1]
    d = user_emb.shape[1]
    bpw = b // NUM_WORKERS
    nch = bpw // CHUNK

    user_idx = x[0]
    movie_idx = x[1]

    mesh = plsc.VectorSubcoreMesh(core_axis_name="c", subcore_axis_name="s")
    run = functools.partial(
        pl.kernel, mesh=mesh,
        out_type=jax.ShapeDtypeStruct((b,), jnp.float32),
        scratch_types=[
            pltpu.VMEM((bpw,), jnp.int32),
            pltpu.VMEM((bpw,), jnp.int32),
            pltpu.VMEM((bpw, d), jnp.float32),
            pltpu.VMEM((bpw, d), jnp.float32),
            pltpu.VMEM((bpw,), jnp.float32),
            pltpu.SemaphoreType.DMA,
        ],
        compiler_params=pltpu.CompilerParams(
            use_tc_tiling_on_sc=False, needs_layout_passes=False),
    )(functools.partial(_dot_kernel, bpw, nch, d))
    return run(user_emb, movie_emb, user_idx, movie_idx)
